# baseline (speedup 1.0000x reference)
import functools

import jax
import jax.numpy as jnp
from jax import lax
from jax.experimental import pallas as pl
from jax.experimental.pallas import tpu as pltpu
from jax.experimental.pallas import tpu_sc as plsc

NC = 2
NS = 16
NW = NC * NS
LANES = 16


@functools.lru_cache(maxsize=None)
def _build_transpose(V, D):
    TW = 384
    NBLK = V // TW
    TAIL = V - NBLK * TW
    assert TAIL % 8 == 0 and TW % 8 == 0
    GMAX = -(-NBLK // NW)
    DV = D // LANES

    mesh = plsc.VectorSubcoreMesh(core_axis_name="c", subcore_axis_name="s")

    @functools.partial(
        pl.kernel,
        out_type=jax.ShapeDtypeStruct((V, D), jnp.float32),
        mesh=mesh,
        compiler_params=pltpu.CompilerParams(
            use_tc_tiling_on_sc=False, needs_layout_passes=False,
        ),
        scratch_types=[
            pltpu.VMEM((2, D, TW), jnp.float32),
            pltpu.VMEM((2, TW, D), jnp.float32),
            [pltpu.SemaphoreType.DMA] * 2,
            [pltpu.SemaphoreType.DMA] * 2,
        ],
    )
    def transpose(src_hbm, dst_hbm, in_v, out_v, isems, osems):
        wid = lax.axis_index("s") * NC + lax.axis_index("c")

        row_idx = [lax.iota(jnp.int32, LANES) + k * LANES for k in range(DV)]

        def blk_of(g):
            return wid + g * NW

        def fire_in(g, s):
            pltpu.async_copy(
                src_hbm.at[:, pl.ds(blk_of(g) * TW, TW)],
                in_v.at[s], isems[s],
            )

        def wait_in(g, s):
            pltpu.make_async_copy(
                src_hbm.at[:, pl.ds(blk_of(g) * TW, TW)],
                in_v.at[s], isems[s],
            ).wait()

        def fire_out(g, s):
            pltpu.async_copy(
                out_v.at[s],
                dst_hbm.at[pl.ds(blk_of(g) * TW, TW)], osems[s],
            )

        def wait_out(g, s):
            pltpu.make_async_copy(
                out_v.at[s],
                dst_hbm.at[pl.ds(blk_of(g) * TW, TW)], osems[s],
            ).wait()

        def shuffle(s):
            def rowbody(r, carry):
                col = jnp.full((LANES,), r, jnp.int32)
                for k in range(DV):
                    out_v[s, r, pl.ds(k * LANES, LANES)] = plsc.load_gather(
                        in_v.at[s], [row_idx[k], col]
                    )
                return carry
            lax.fori_loop(0, TW, rowbody, 0, unroll=4)

        nb = (NBLK - wid + NW - 1) // NW

        for s in range(2):
            @pl.when(s < nb)
            def _():
                fire_in(s, s)

        def outer(h, carry):
            for s in range(2):
                g = h * 2 + s

                @pl.when(g < nb)
                def _():
                    wait_in(g, s)

                    @pl.when(g >= 2)
                    def _():
                        wait_out(g - 2, s)

                    shuffle(s)
                    fire_out(g, s)

                    @pl.when(g + 2 < nb)
                    def _():
                        fire_in(g + 2, s)
            return carry

        lax.fori_loop(0, (GMAX + 1) // 2, outer, 0)

        for s in range(2):
            @pl.when(nb > s)
            def _():
                g_last = ((nb - 1 - s) // 2) * 2 + s
                wait_out(g_last, s)

        if TAIL:
            @pl.when(wid == 0)
            def _():
                pltpu.sync_copy(
                    src_hbm.at[:, pl.ds(NBLK * TW, TAIL)],
                    in_v.at[0, :, pl.ds(0, TAIL)],
                )

                def rowbody(r, carry):
                    col = jnp.full((LANES,), r, jnp.int32)
                    for k in range(DV):
                        out_v[0, r, pl.ds(k * LANES, LANES)] = plsc.load_gather(
                            in_v.at[0], [row_idx[k], col]
                        )
                    return carry
                lax.fori_loop(0, TAIL, rowbody, 0)
                pltpu.sync_copy(
                    out_v.at[0, pl.ds(0, TAIL)],
                    dst_hbm.at[pl.ds(NBLK * TW, TAIL)],
                )

    return transpose


@functools.lru_cache(maxsize=None)
def _build_encoder(B, L, V, D):
    EPW = B // NW
    NCH = -(-L // 128)
    assert L % NCH == 0
    CH = L // NCH
    DV = D // LANES
    NBUF = 4
    ROWS_PER_W = EPW * NCH

    mesh = plsc.VectorSubcoreMesh(core_axis_name="c", subcore_axis_name="s")

    @functools.partial(
        pl.kernel,
        out_type=jax.ShapeDtypeStruct((B, D), jnp.float32),
        mesh=mesh,
        compiler_params=pltpu.CompilerParams(use_tc_tiling_on_sc=False),
        scratch_types=[
            pltpu.VMEM((ROWS_PER_W, CH), jnp.int32),
            pltpu.VMEM((NBUF, L, D), jnp.float32),
            pltpu.VMEM((EPW, D), jnp.float32),
            [pltpu.SemaphoreType.DMA] * NBUF,
        ],
    )
    def encoder(tok_hbm, table_hbm, out_hbm, idx_v, rows_v, out_v, sems):
        wid = lax.axis_index("s") * NC + lax.axis_index("c")
        base = wid * EPW

        pltpu.sync_copy(tok_hbm.at[pl.ds(wid * ROWS_PER_W, ROWS_PER_W)], idx_v)

        def fire(e, b):
            for c in range(NCH):
                pltpu.async_copy(
                    table_hbm.at[idx_v.at[e * NCH + c]],
                    rows_v.at[b, pl.ds(c * CH, CH)],
                    sems[b],
                )

        def drain(e, b):
            for c in range(NCH):
                pltpu.make_async_copy(
                    table_hbm.at[idx_v.at[e * NCH + c]],
                    rows_v.at[b, pl.ds(c * CH, CH)],
                    sems[b],
                ).wait()

        for b in range(NBUF):
            fire(b, b)

        inv_l = jnp.float32(1.0 / L)

        def reduce_elem(e, b):
            def body(j, accs):
                return tuple(
                    a + rows_v[b, j, pl.ds(k * LANES, LANES)]
                    for k, a in enumerate(accs)
                )
            accs = lax.fori_loop(
                0, L, body,
                tuple(jnp.zeros((LANES,), jnp.float32) for _ in range(DV)),
                unroll=8,
            )
            for k in range(DV):
                out_v[e, pl.ds(k * LANES, LANES)] = accs[k] * inv_l

        def outer(g, carry):
            for b in range(NBUF):
                e = g * NBUF + b
                drain(e, b)
                reduce_elem(e, b)

                @pl.when(e + NBUF < EPW)
                def _():
                    fire(e + NBUF, b)
            return carry

        lax.fori_loop(0, EPW // NBUF, outer, 0)

        pltpu.sync_copy(out_v, out_hbm.at[pl.ds(base, EPW)])

    return encoder


def kernel(token_ids, table):
    B, L = token_ids.shape
    V, D = table.shape
    NCH = -(-L // 128)
    tok = token_ids.astype(jnp.int32).reshape(B * NCH, L // NCH)
    table_rm = _build_transpose(V, D)(table.T)
    return _build_encoder(B, L, V, D)(tok, table_rm)


# speedup vs baseline: 4.5802x; 4.5802x over previous
import functools

import jax
import jax.numpy as jnp
from jax import lax
from jax.experimental import pallas as pl
from jax.experimental.pallas import tpu as pltpu
from jax.experimental.pallas import tpu_sc as plsc

NC = 2
NS = 16
NW = NC * NS
LANES = 16


@functools.lru_cache(maxsize=None)
def _build_transpose(V, D):
    TW = 384
    NBLK = (V // TW)
    TAIL = V - NBLK * TW
    GMAX = -(-NBLK // NW)
    TW2 = TW // 2
    PD = 2 * D
    QN = TW // LANES

    mesh = plsc.VectorSubcoreMesh(core_axis_name="c", subcore_axis_name="s")

    @functools.partial(
        pl.kernel,
        out_type=jax.ShapeDtypeStruct((V // 2, PD), jnp.float32),
        mesh=mesh,
        compiler_params=pltpu.CompilerParams(
            use_tc_tiling_on_sc=True, needs_layout_passes=False,
        ),
        scratch_types=[
            pltpu.VMEM((2, D, TW), jnp.float32),
            pltpu.VMEM((2, TW2, PD), jnp.float32),
            pltpu.VMEM((D, TAIL if TAIL else LANES), jnp.float32),
            [pltpu.SemaphoreType.DMA] * 2,
            [pltpu.SemaphoreType.DMA] * 2,
        ],
    )
    def transpose(src_hbm, dst_hbm, in_v, out_v, tail_v, isems, osems):
        wid = lax.axis_index("s") * NC + lax.axis_index("c")

        iota = lax.iota(jnp.int32, LANES)
        rowpat = lax.shift_right_logical(iota, 1)
        colpat = lax.bitwise_and(iota, 1) * D

        def blk_of(g):
            return wid + g * NW

        def fire_in(g, s):
            pltpu.async_copy(
                src_hbm.at[:, pl.ds(blk_of(g) * TW, TW)],
                in_v.at[s], isems[s],
            )

        def wait_in(g, s):
            pltpu.make_async_copy(
                src_hbm.at[:, pl.ds(blk_of(g) * TW, TW)],
                in_v.at[s], isems[s],
            ).wait()

        def fire_out(g, s):
            pltpu.async_copy(
                out_v.at[s],
                dst_hbm.at[pl.ds(blk_of(g) * TW2, TW2)], osems[s],
            )

        def wait_out(g, s):
            pltpu.make_async_copy(
                out_v.at[s],
                dst_hbm.at[pl.ds(blk_of(g) * TW2, TW2)], osems[s],
            ).wait()

        def shuffle(s, qn):
            def dbody(d, carry):
                for q in range(qn):
                    vals = in_v[s, d, pl.ds(q * LANES, LANES)]
                    plsc.store_scatter(
                        out_v.at[s],
                        [rowpat + (8 * q), colpat + d],
                        vals,
                    )
                return carry
            lax.fori_loop(0, D, dbody, 0)

        nb = (NBLK - wid + NW - 1) // NW

        for s in range(2):
            @pl.when(s < nb)
            def _():
                fire_in(s, s)

        def outer(h, carry):
            for s in range(2):
                g = h * 2 + s

                @pl.when(g < nb)
                def _():
                    wait_in(g, s)

                    @pl.when(g >= 2)
                    def _():
                        wait_out(g - 2, s)

                    shuffle(s, QN)
                    fire_out(g, s)

                    @pl.when(g + 2 < nb)
                    def _():
                        fire_in(g + 2, s)
            return carry

        lax.fori_loop(0, (GMAX + 1) // 2, outer, 0)

        for s in range(2):
            @pl.when(nb > s)
            def _():
                g_last = ((nb - 1 - s) // 2) * 2 + s
                wait_out(g_last, s)

        if TAIL:
            @pl.when(wid == 0)
            def _():
                pltpu.sync_copy(
                    src_hbm.at[:, pl.ds(NBLK * TW, TAIL)],
                    tail_v,
                )

                def dbody(d, carry):
                    for q in range(TAIL // LANES):
                        vals = tail_v[d, pl.ds(q * LANES, LANES)]
                        plsc.store_scatter(
                            out_v.at[0],
                            [rowpat + (8 * q), colpat + d],
                            vals,
                        )
                    return carry
                lax.fori_loop(0, D, dbody, 0)
                pltpu.sync_copy(
                    out_v.at[0, pl.ds(0, TAIL // 2)],
                    dst_hbm.at[pl.ds(NBLK * TW // 2, TAIL // 2)],
                )

    return transpose


@functools.lru_cache(maxsize=None)
def _build_encoder(B, L, V2, PD):
    D = PD // 2
    EPW = B // NW
    TPW = EPW * L
    DV = D // LANES
    CH0 = 104
    CH1 = L - CH0
    NBUF = 2

    mesh = plsc.VectorSubcoreMesh(core_axis_name="c", subcore_axis_name="s")

    @functools.partial(
        pl.kernel,
        out_type=jax.ShapeDtypeStruct((B, D), jnp.float32),
        mesh=mesh,
        compiler_params=pltpu.CompilerParams(use_tc_tiling_on_sc=True),
        scratch_types=[
            pltpu.VMEM((TPW + LANES,), jnp.int32),
            pltpu.VMEM((TPW,), jnp.int32),
            pltpu.VMEM((NBUF, L, PD), jnp.float32),
            pltpu.VMEM((EPW, D), jnp.float32),
            [pltpu.SemaphoreType.DMA] * NBUF,
        ],
    )
    def encoder(tok_hbm, table_hbm, out_hbm, tok_v, idx_v, rows_v, out_v, sems):
        wid = lax.axis_index("s") * NC + lax.axis_index("c")
        base = wid * EPW

        pltpu.sync_copy(tok_hbm.at[wid], tok_v.at[pl.ds(0, TPW)])

        def prep(k, carry):
            t = tok_v[pl.ds(k * LANES, LANES)]
            idx_v[pl.ds(k * LANES, LANES)] = lax.shift_right_logical(t, 1)
            tok_v[pl.ds(k * LANES, LANES)] = lax.bitwise_and(t, 1)
            return carry

        lax.fori_loop(0, TPW // LANES, prep, 0, unroll=8)

        def fire(e, b):
            pltpu.async_copy(
                table_hbm.at[idx_v.at[pl.ds(e * L, CH0)]],
                rows_v.at[b, pl.ds(0, CH0)],
                sems[b],
            )
            pltpu.async_copy(
                table_hbm.at[idx_v.at[pl.ds(e * L + CH0, CH1)]],
                rows_v.at[b, pl.ds(CH0, CH1)],
                sems[b],
            )

        def drain(e, b):
            pltpu.make_async_copy(
                table_hbm.at[idx_v.at[pl.ds(e * L, CH0)]],
                rows_v.at[b, pl.ds(0, CH0)],
                sems[b],
            ).wait()
            pltpu.make_async_copy(
                table_hbm.at[idx_v.at[pl.ds(e * L + CH0, CH1)]],
                rows_v.at[b, pl.ds(CH0, CH1)],
                sems[b],
            ).wait()

        for b in range(NBUF):
            fire(b, b)

        inv_l = jnp.float32(1.0 / L)
        NG = L // LANES
        TAILG = L - NG * LANES

        def reduce_elem(e, b):
            def group(g, accs, cnt):
                hv = tok_v[pl.ds(e * L + g * LANES, LANES)]
                for jj in range(cnt):
                    half = hv[jj] * D
                    j = g * LANES + jj
                    accs = tuple(
                        a + rows_v[b, j, pl.ds(half + k * LANES, LANES)]
                        for k, a in enumerate(accs)
                    )
                return accs

            init = tuple(jnp.zeros((LANES,), jnp.float32) for _ in range(DV))
            accs = lax.fori_loop(
                0, NG, lambda g, accs: group(g, accs, LANES), init,
            )
            if TAILG:
                accs = group(NG, accs, TAILG)
            for k in range(DV):
                out_v[e, pl.ds(k * LANES, LANES)] = accs[k] * inv_l

        def outer(g, carry):
            for b in range(NBUF):
                e = g * NBUF + b
                drain(e, b)
                reduce_elem(e, b)

                @pl.when(e + NBUF < EPW)
                def _():
                    fire(e + NBUF, b)
            return carry

        lax.fori_loop(0, EPW // NBUF, outer, 0)

        pltpu.sync_copy(out_v, out_hbm.at[pl.ds(base, EPW)])

    return encoder


def kernel(token_ids, table):
    B, L = token_ids.shape
    V, D = table.shape
    tok = token_ids.astype(jnp.int32).reshape(NW, (B // NW) * L)
    pair_table = _build_transpose(V, D)(table.T)
    return _build_encoder(B, L, V // 2, 2 * D)(tok, pair_table)


# speedup vs baseline: 6.0794x; 1.3273x over previous
import functools

import jax
import jax.numpy as jnp
from jax import lax
from jax.experimental import pallas as pl
from jax.experimental.pallas import tpu as pltpu
from jax.experimental.pallas import tpu_sc as plsc

NC = 2
NS = 16
NW = NC * NS
LANES = 16


@functools.lru_cache(maxsize=None)
def _build_transpose(V, D):
    TW = 384
    NBLK = (V // TW)
    TAIL = V - NBLK * TW
    GMAX = -(-NBLK // NW)
    TW2 = TW // 2
    PD = 2 * D
    QN = TW // LANES

    mesh = plsc.VectorSubcoreMesh(core_axis_name="c", subcore_axis_name="s")

    @functools.partial(
        pl.kernel,
        out_type=jax.ShapeDtypeStruct((V // 2, PD), jnp.float32),
        mesh=mesh,
        compiler_params=pltpu.CompilerParams(
            use_tc_tiling_on_sc=True, needs_layout_passes=False,
        ),
        scratch_types=[
            pltpu.VMEM((2, D, TW), jnp.float32),
            pltpu.VMEM((2, TW2, PD), jnp.float32),
            pltpu.VMEM((D, TAIL if TAIL else LANES), jnp.float32),
            [pltpu.SemaphoreType.DMA] * 2,
            [pltpu.SemaphoreType.DMA] * 2,
        ],
    )
    def transpose(src_hbm, dst_hbm, in_v, out_v, tail_v, isems, osems):
        wid = lax.axis_index("s") * NC + lax.axis_index("c")

        iota = lax.iota(jnp.int32, LANES)
        rowpat = lax.shift_right_logical(iota, 1)
        colpat = lax.bitwise_and(iota, 1) * D

        def blk_of(g):
            return wid + g * NW

        def fire_in(g, s):
            pltpu.async_copy(
                src_hbm.at[:, pl.ds(blk_of(g) * TW, TW)],
                in_v.at[s], isems[s],
            )

        def wait_in(g, s):
            pltpu.make_async_copy(
                src_hbm.at[:, pl.ds(blk_of(g) * TW, TW)],
                in_v.at[s], isems[s],
            ).wait()

        def fire_out(g, s):
            pltpu.async_copy(
                out_v.at[s],
                dst_hbm.at[pl.ds(blk_of(g) * TW2, TW2)], osems[s],
            )

        def wait_out(g, s):
            pltpu.make_async_copy(
                out_v.at[s],
                dst_hbm.at[pl.ds(blk_of(g) * TW2, TW2)], osems[s],
            ).wait()

        def shuffle(s, qn):
            @plsc.parallel_loop(0, D, unroll=4)
            def dbody(d):
                colv = colpat + d
                for q in range(qn):
                    vals = in_v[s, d, pl.ds(q * LANES, LANES)]
                    plsc.store_scatter(
                        out_v.at[s],
                        [rowpat + (8 * q), colv],
                        vals,
                    )

        nb = (NBLK - wid + NW - 1) // NW

        for s in range(2):
            @pl.when(s < nb)
            def _():
                fire_in(s, s)

        def outer(h, carry):
            for s in range(2):
                g = h * 2 + s

                @pl.when(g < nb)
                def _():
                    wait_in(g, s)

                    @pl.when(g >= 2)
                    def _():
                        wait_out(g - 2, s)

                    shuffle(s, QN)
                    fire_out(g, s)

                    @pl.when(g + 2 < nb)
                    def _():
                        fire_in(g + 2, s)
            return carry

        lax.fori_loop(0, (GMAX + 1) // 2, outer, 0)

        for s in range(2):
            @pl.when(nb > s)
            def _():
                g_last = ((nb - 1 - s) // 2) * 2 + s
                wait_out(g_last, s)

        if TAIL:
            @pl.when(wid == 0)
            def _():
                pltpu.sync_copy(
                    src_hbm.at[:, pl.ds(NBLK * TW, TAIL)],
                    tail_v,
                )

                def dbody(d, carry):
                    for q in range(TAIL // LANES):
                        vals = tail_v[d, pl.ds(q * LANES, LANES)]
                        plsc.store_scatter(
                            out_v.at[0],
                            [rowpat + (8 * q), colpat + d],
                            vals,
                        )
                    return carry
                lax.fori_loop(0, D, dbody, 0)
                pltpu.sync_copy(
                    out_v.at[0, pl.ds(0, TAIL // 2)],
                    dst_hbm.at[pl.ds(NBLK * TW // 2, TAIL // 2)],
                )

    return transpose


@functools.lru_cache(maxsize=None)
def _build_encoder(B, L, V2, PD):
    D = PD // 2
    EPW = B // NW
    TPW = EPW * L
    DV = D // LANES
    CH0 = 104
    CH1 = L - CH0
    NBUF = 2

    mesh = plsc.VectorSubcoreMesh(core_axis_name="c", subcore_axis_name="s")

    @functools.partial(
        pl.kernel,
        out_type=jax.ShapeDtypeStruct((B, D), jnp.float32),
        mesh=mesh,
        compiler_params=pltpu.CompilerParams(use_tc_tiling_on_sc=True),
        scratch_types=[
            pltpu.VMEM((TPW + LANES,), jnp.int32),
            pltpu.VMEM((TPW,), jnp.int32),
            pltpu.VMEM((NBUF, L, PD), jnp.float32),
            pltpu.VMEM((EPW, D), jnp.float32),
            [pltpu.SemaphoreType.DMA] * NBUF,
        ],
    )
    def encoder(tok_hbm, table_hbm, out_hbm, tok_v, idx_v, rows_v, out_v, sems):
        wid = lax.axis_index("s") * NC + lax.axis_index("c")
        base = wid * EPW

        pltpu.sync_copy(tok_hbm.at[wid], tok_v.at[pl.ds(0, TPW)])

        def prep(k, carry):
            t = tok_v[pl.ds(k * LANES, LANES)]
            idx_v[pl.ds(k * LANES, LANES)] = lax.shift_right_logical(t, 1)
            tok_v[pl.ds(k * LANES, LANES)] = lax.bitwise_and(t, 1)
            return carry

        lax.fori_loop(0, TPW // LANES, prep, 0, unroll=8)

        def fire(e, b):
            pltpu.async_copy(
                table_hbm.at[idx_v.at[pl.ds(e * L, CH0)]],
                rows_v.at[b, pl.ds(0, CH0)],
                sems[b],
            )
            pltpu.async_copy(
                table_hbm.at[idx_v.at[pl.ds(e * L + CH0, CH1)]],
                rows_v.at[b, pl.ds(CH0, CH1)],
                sems[b],
            )

        def drain(e, b):
            pltpu.make_async_copy(
                table_hbm.at[idx_v.at[pl.ds(e * L, CH0)]],
                rows_v.at[b, pl.ds(0, CH0)],
                sems[b],
            ).wait()
            pltpu.make_async_copy(
                table_hbm.at[idx_v.at[pl.ds(e * L + CH0, CH1)]],
                rows_v.at[b, pl.ds(CH0, CH1)],
                sems[b],
            ).wait()

        for b in range(NBUF):
            fire(b, b)

        inv_l = jnp.float32(1.0 / L)
        NG = L // LANES
        TAILG = L - NG * LANES

        def reduce_elem(e, b):
            def group(g, accs, cnt):
                hv = tok_v[pl.ds(e * L + g * LANES, LANES)]
                for jj in range(cnt):
                    half = hv[jj] * D
                    j = g * LANES + jj
                    accs = tuple(
                        a + rows_v[b, j, pl.ds(half + k * LANES, LANES)]
                        for k, a in enumerate(accs)
                    )
                return accs

            init = tuple(jnp.zeros((LANES,), jnp.float32) for _ in range(DV))
            accs = lax.fori_loop(
                0, NG, lambda g, accs: group(g, accs, LANES), init,
            )
            if TAILG:
                accs = group(NG, accs, TAILG)
            for k in range(DV):
                out_v[e, pl.ds(k * LANES, LANES)] = accs[k] * inv_l

        def outer(g, carry):
            for b in range(NBUF):
                e = g * NBUF + b
                drain(e, b)
                reduce_elem(e, b)

                @pl.when(e + NBUF < EPW)
                def _():
                    fire(e + NBUF, b)
            return carry

        lax.fori_loop(0, EPW // NBUF, outer, 0)

        pltpu.sync_copy(out_v, out_hbm.at[pl.ds(base, EPW)])

    return encoder


def kernel(token_ids, table):
    B, L = token_ids.shape
    V, D = table.shape
    tok = token_ids.astype(jnp.int32).reshape(NW, (B // NW) * L)
    pair_table = _build_transpose(V, D)(table.T)
    return _build_encoder(B, L, V // 2, 2 * D)(tok, pair_table)
